# baseline (speedup 1.0000x reference)
import functools

import jax
import jax.numpy as jnp
from jax import lax
from jax.experimental import pallas as pl
from jax.experimental.pallas import tpu as pltpu

N = 5000
NP = 5120
K = 1000
KP = 1024
SJ = 32
TN = 512
RB = 128
NB = 32
IOU_THRESHOLD = 0.5
WIN_Y = 512.0
WIN_X = 512.0


def _nms_body(sr_ref, sc_ref, bd_ref, out_ref, s_ref):
    sr = sr_ref[...]
    b = bd_ref[...]

    y1, x1, y2, x2 = b[0:1], b[1:2], b[2:3], b[3:4]
    dy, dx, dh, dw = b[4:5], b[5:6], b[6:7], b[7:8]
    h = y2 - y1
    w = x2 - x1
    cy = y1 + 0.5 * h
    cx = x1 + 0.5 * w
    pcy = dy * h + cy
    pcx = dx * w + cx
    ph = jnp.exp(dh) * h
    pw = jnp.exp(dw) * w
    py1 = jnp.clip(pcy - 0.5 * ph, 0.0, WIN_Y)
    px1 = jnp.clip(pcx - 0.5 * pw, 0.0, WIN_X)
    py2 = jnp.clip(pcy + 0.5 * ph, 0.0, WIN_Y)
    px2 = jnp.clip(pcx + 0.5 * pw, 0.0, WIN_X)
    pred5 = jnp.concatenate(
        [py1, px1, py2, px2, sr, jnp.zeros((3, NP), jnp.float32)], axis=0
    )

    liota = lax.broadcasted_iota(jnp.int32, (1, NP), 1)

    def rank_body(jt, acc):
        sct = sc_ref[pl.ds(jt * SJ, SJ), :]
        jidx = jt * SJ + lax.broadcasted_iota(jnp.int32, (SJ, 1), 0)
        beats = (sct > sr) | ((sct == sr) & (jidx < liota))
        return acc + jnp.sum(
            jnp.where(beats, 1.0, 0.0), axis=0, keepdims=True)

    rank = lax.fori_loop(0, NP // SJ, rank_body,
                         jnp.zeros((1, NP), jnp.float32))

    kio = lax.broadcasted_iota(jnp.int32, (KP, 1), 0).astype(jnp.float32)
    dn = (((1,), (1,)), ((), ()))
    sel_r = jnp.zeros((8, KP), jnp.float32)
    sel_c = jnp.zeros((KP, 8), jnp.float32)
    for t in range(NP // TN):
        rk = rank[:, t * TN:(t + 1) * TN]
        oh = jnp.where(rk == kio, 1.0, 0.0)
        pr = pred5[:, t * TN:(t + 1) * TN]
        sel_r = sel_r + lax.dot_general(
            pr, oh, dn, precision=lax.Precision.HIGHEST,
            preferred_element_type=jnp.float32)
        sel_c = sel_c + lax.dot_general(
            oh, pr, dn, precision=lax.Precision.HIGHEST,
            preferred_element_type=jnp.float32)

    def to88(v):
        return jnp.concatenate(
            [v[:, s * 128:(s + 1) * 128] for s in range(8)],
            axis=0).reshape(1, 8, 128)

    y1r, x1r, y2r, x2r = (to88(sel_r[c:c + 1]) for c in range(4))
    area_r = jnp.maximum(y2r - y1r, 0.0) * jnp.maximum(x2r - x1r, 0.0)
    lio3 = (lax.broadcasted_iota(jnp.int32, (1, 8, 128), 1) * 128
            + lax.broadcasted_iota(jnp.int32, (1, 8, 128), 2))
    for rb in range(KP // RB):
        sl = slice(rb * RB, (rb + 1) * RB)
        y1c = sel_c[sl, 0:1].reshape(RB, 1, 1)
        x1c = sel_c[sl, 1:2].reshape(RB, 1, 1)
        y2c = sel_c[sl, 2:3].reshape(RB, 1, 1)
        x2c = sel_c[sl, 3:4].reshape(RB, 1, 1)
        area_c = jnp.maximum(y2c - y1c, 0.0) * jnp.maximum(x2c - x1c, 0.0)
        yy1 = jnp.maximum(y1c, y1r)
        xx1 = jnp.maximum(x1c, x1r)
        yy2 = jnp.minimum(y2c, y2r)
        xx2 = jnp.minimum(x2c, x2r)
        inter = jnp.maximum(yy2 - yy1, 0.0) * jnp.maximum(xx2 - xx1, 0.0)
        union = area_c + area_r - inter
        iou = inter / (union + 1e-6)
        sio = lax.broadcasted_iota(jnp.int32, (RB, 1, 1), 0) + rb * RB
        s_ref[sl] = jnp.where((iou > IOU_THRESHOLD) & (lio3 > sio), 1.0, 0.0)

    iota88 = lio3.reshape(8, 128)

    def nms_step(i, keep):
        row = s_ref[pl.ds(i, 1)].reshape(8, 128)
        ki = jnp.sum(jnp.where(iota88 == i, keep, 0.0), axis=(0, 1),
                     keepdims=True)
        return keep * (1.0 - row * ki)

    keep88 = lax.fori_loop(0, K, nms_step, jnp.ones((8, 128), jnp.float32))
    keep = jnp.concatenate([keep88[s:s + 1, :] for s in range(8)], axis=1)
    out_ref[...] = sel_r * keep


@jax.jit
def kernel(boxes, deltas, scores):
    sp = jnp.pad(scores, (0, NP - N), constant_values=-1.0)
    sr = sp.reshape(1, NP)
    sc = sp.reshape(NP, 1)
    bd = jnp.concatenate(
        [jnp.pad(boxes, ((0, NP - N), (0, 0))).T,
         jnp.pad(deltas, ((0, NP - N), (0, 0))).T], axis=0)

    out = pl.pallas_call(
        _nms_body,
        out_shape=jax.ShapeDtypeStruct((8, KP), jnp.float32),
        scratch_shapes=[pltpu.VMEM((KP, 8, 128), jnp.float32)],
    )(sr, sc, bd)
    return out.T[:K, :5]


# speedup vs baseline: 1.0086x; 1.0086x over previous
import functools

import jax
import jax.numpy as jnp
from jax import lax
from jax.experimental import pallas as pl
from jax.experimental.pallas import tpu as pltpu

N = 5000
NP = 5120
K = 1000
KP = 1024
SJ = 32
TN = 512
RB = 128
NB = 32
IOU_THRESHOLD = 0.5
WIN_Y = 512.0
WIN_X = 512.0


def _nms_body(sr_ref, sc_ref, bd_ref, out_ref, s_ref):
    sr = sr_ref[...]
    b = bd_ref[...]

    y1, x1, y2, x2 = b[0:1], b[1:2], b[2:3], b[3:4]
    dy, dx, dh, dw = b[4:5], b[5:6], b[6:7], b[7:8]
    h = y2 - y1
    w = x2 - x1
    cy = y1 + 0.5 * h
    cx = x1 + 0.5 * w
    pcy = dy * h + cy
    pcx = dx * w + cx
    ph = jnp.exp(dh) * h
    pw = jnp.exp(dw) * w
    py1 = jnp.clip(pcy - 0.5 * ph, 0.0, WIN_Y)
    px1 = jnp.clip(pcx - 0.5 * pw, 0.0, WIN_X)
    py2 = jnp.clip(pcy + 0.5 * ph, 0.0, WIN_Y)
    px2 = jnp.clip(pcx + 0.5 * pw, 0.0, WIN_X)
    pred5 = jnp.concatenate(
        [py1, px1, py2, px2, sr, jnp.zeros((3, NP), jnp.float32)], axis=0
    )

    liota = lax.broadcasted_iota(jnp.int32, (1, NP), 1)

    def rank_body(jt, acc):
        sct = sc_ref[pl.ds(jt * SJ, SJ), :]
        jidx = jt * SJ + lax.broadcasted_iota(jnp.int32, (SJ, 1), 0)
        beats = (sct > sr) | ((sct == sr) & (jidx < liota))
        return acc + jnp.sum(
            jnp.where(beats, 1.0, 0.0), axis=0, keepdims=True)

    rank = lax.fori_loop(0, NP // SJ, rank_body,
                         jnp.zeros((1, NP), jnp.float32))

    kio = lax.broadcasted_iota(jnp.int32, (KP, 1), 0).astype(jnp.float32)
    dn = (((1,), (1,)), ((), ()))
    sel_r = jnp.zeros((8, KP), jnp.float32)
    sel_c = jnp.zeros((KP, 8), jnp.float32)
    for t in range(NP // TN):
        rk = rank[:, t * TN:(t + 1) * TN]
        oh = jnp.where(rk == kio, 1.0, 0.0)
        pr = pred5[:, t * TN:(t + 1) * TN]
        sel_r = sel_r + lax.dot_general(
            pr, oh, dn, precision=lax.Precision.HIGHEST,
            preferred_element_type=jnp.float32)
        sel_c = sel_c + lax.dot_general(
            oh, pr, dn, precision=lax.Precision.HIGHEST,
            preferred_element_type=jnp.float32)

    def to88(v):
        return jnp.concatenate(
            [v[:, s * 128:(s + 1) * 128] for s in range(8)],
            axis=0).reshape(1, 8, 128)

    y1r, x1r, y2r, x2r = (to88(sel_r[c:c + 1]) for c in range(4))
    area_r = jnp.maximum(y2r - y1r, 0.0) * jnp.maximum(x2r - x1r, 0.0)
    lio3 = (lax.broadcasted_iota(jnp.int32, (1, 8, 128), 1) * 128
            + lax.broadcasted_iota(jnp.int32, (1, 8, 128), 2))
    for rb in range(KP // RB):
        sl = slice(rb * RB, (rb + 1) * RB)
        y1c = sel_c[sl, 0:1].reshape(RB, 1, 1)
        x1c = sel_c[sl, 1:2].reshape(RB, 1, 1)
        y2c = sel_c[sl, 2:3].reshape(RB, 1, 1)
        x2c = sel_c[sl, 3:4].reshape(RB, 1, 1)
        area_c = jnp.maximum(y2c - y1c, 0.0) * jnp.maximum(x2c - x1c, 0.0)
        yy1 = jnp.maximum(y1c, y1r)
        xx1 = jnp.maximum(x1c, x1r)
        yy2 = jnp.minimum(y2c, y2r)
        xx2 = jnp.minimum(x2c, x2r)
        inter = jnp.maximum(yy2 - yy1, 0.0) * jnp.maximum(xx2 - xx1, 0.0)
        union = area_c + area_r - inter
        iou = inter / (union + 1e-6)
        sio = lax.broadcasted_iota(jnp.int32, (RB, 1, 1), 0) + rb * RB
        s_ref[sl] = jnp.where((iou > IOU_THRESHOLD) & (lio3 > sio), 1.0, 0.0)

    iota88 = lio3.reshape(8, 128)

    def nms_step(i, keep):
        row = s_ref[pl.ds(i, 1)].reshape(8, 128)
        ki = jnp.sum(jnp.where(iota88 == i, keep, 0.0), axis=(0, 1),
                     keepdims=True)
        return keep * (1.0 - row * ki)

    keep88 = lax.fori_loop(0, K, nms_step, jnp.ones((8, 128), jnp.float32),
                           unroll=8)
    keep = jnp.concatenate([keep88[s:s + 1, :] for s in range(8)], axis=1)
    out_ref[...] = sel_r * keep


@jax.jit
def kernel(boxes, deltas, scores):
    sp = jnp.pad(scores, (0, NP - N), constant_values=-1.0)
    sr = sp.reshape(1, NP)
    sc = sp.reshape(NP, 1)
    bd = jnp.concatenate(
        [jnp.pad(boxes, ((0, NP - N), (0, 0))).T,
         jnp.pad(deltas, ((0, NP - N), (0, 0))).T], axis=0)

    out = pl.pallas_call(
        _nms_body,
        out_shape=jax.ShapeDtypeStruct((8, KP), jnp.float32),
        scratch_shapes=[pltpu.VMEM((KP, 8, 128), jnp.float32)],
    )(sr, sc, bd)
    return out.T[:K, :5]


# speedup vs baseline: 1.8052x; 1.7897x over previous
import functools

import jax
import jax.numpy as jnp
from jax import lax
from jax.experimental import pallas as pl
from jax.experimental.pallas import tpu as pltpu

N = 5000
NP = 5120
K = 1000
KP = 1024
SJ = 32
TN = 512
RB = 128
NB = 32
IOU_THRESHOLD = 0.5
WIN_Y = 512.0
WIN_X = 512.0


def _nms_body(sr_ref, sc_ref, bd_ref, out_ref, s_ref):
    sr = sr_ref[...]
    b = bd_ref[...]

    y1, x1, y2, x2 = b[0:1], b[1:2], b[2:3], b[3:4]
    dy, dx, dh, dw = b[4:5], b[5:6], b[6:7], b[7:8]
    h = y2 - y1
    w = x2 - x1
    cy = y1 + 0.5 * h
    cx = x1 + 0.5 * w
    pcy = dy * h + cy
    pcx = dx * w + cx
    ph = jnp.exp(dh) * h
    pw = jnp.exp(dw) * w
    py1 = jnp.clip(pcy - 0.5 * ph, 0.0, WIN_Y)
    px1 = jnp.clip(pcx - 0.5 * pw, 0.0, WIN_X)
    py2 = jnp.clip(pcy + 0.5 * ph, 0.0, WIN_Y)
    px2 = jnp.clip(pcx + 0.5 * pw, 0.0, WIN_X)
    pred5 = jnp.concatenate(
        [py1, px1, py2, px2, sr, jnp.zeros((3, NP), jnp.float32)], axis=0
    )

    liota = lax.broadcasted_iota(jnp.int32, (1, NP), 1)

    def rank_body(jt, acc):
        sct = sc_ref[pl.ds(jt * SJ, SJ), :]
        jidx = jt * SJ + lax.broadcasted_iota(jnp.int32, (SJ, 1), 0)
        beats = (sct > sr) | ((sct == sr) & (jidx < liota))
        return acc + jnp.sum(
            jnp.where(beats, 1.0, 0.0), axis=0, keepdims=True)

    rank = lax.fori_loop(0, NP // SJ, rank_body,
                         jnp.zeros((1, NP), jnp.float32))

    kio = lax.broadcasted_iota(jnp.int32, (KP, 1), 0).astype(jnp.float32)
    dn = (((1,), (1,)), ((), ()))
    sel_r = jnp.zeros((8, KP), jnp.float32)
    sel_c = jnp.zeros((KP, 8), jnp.float32)
    for t in range(NP // TN):
        rk = rank[:, t * TN:(t + 1) * TN]
        oh = jnp.where(rk == kio, 1.0, 0.0)
        pr = pred5[:, t * TN:(t + 1) * TN]
        sel_r = sel_r + lax.dot_general(
            pr, oh, dn, precision=lax.Precision.HIGHEST,
            preferred_element_type=jnp.float32)
        sel_c = sel_c + lax.dot_general(
            oh, pr, dn, precision=lax.Precision.HIGHEST,
            preferred_element_type=jnp.float32)

    def to88(v):
        return jnp.concatenate(
            [v[:, s * 128:(s + 1) * 128] for s in range(8)],
            axis=0).reshape(1, 8, 128)

    y1r, x1r, y2r, x2r = (to88(sel_r[c:c + 1]) for c in range(4))
    area_r = jnp.maximum(y2r - y1r, 0.0) * jnp.maximum(x2r - x1r, 0.0)
    lio3 = (lax.broadcasted_iota(jnp.int32, (1, 8, 128), 1) * 128
            + lax.broadcasted_iota(jnp.int32, (1, 8, 128), 2))
    for rb in range(KP // RB):
        sl = slice(rb * RB, (rb + 1) * RB)
        y1c = sel_c[sl, 0:1].reshape(RB, 1, 1)
        x1c = sel_c[sl, 1:2].reshape(RB, 1, 1)
        y2c = sel_c[sl, 2:3].reshape(RB, 1, 1)
        x2c = sel_c[sl, 3:4].reshape(RB, 1, 1)
        area_c = jnp.maximum(y2c - y1c, 0.0) * jnp.maximum(x2c - x1c, 0.0)
        yy1 = jnp.maximum(y1c, y1r)
        xx1 = jnp.maximum(x1c, x1r)
        yy2 = jnp.minimum(y2c, y2r)
        xx2 = jnp.minimum(x2c, x2r)
        inter = jnp.maximum(yy2 - yy1, 0.0) * jnp.maximum(xx2 - xx1, 0.0)
        union = area_c + area_r - inter
        iou = inter / (union + 1e-6)
        sio = lax.broadcasted_iota(jnp.int32, (RB, 1, 1), 0) + rb * RB
        s_ref[sl] = jnp.where((iou > IOU_THRESHOLD) & (lio3 > sio), 1.0, 0.0)

    iota88 = lio3.reshape(8, 128)
    U = 8

    def nms_group(g, keep):
        base = g * U
        rows3 = s_ref[pl.ds(base, U)]
        rows = [rows3[u].reshape(8, 128) for u in range(U)]
        masks = [iota88 == (base + u) for u in range(U)]
        ks = [jnp.sum(jnp.where(masks[u], keep, 0.0), axis=(0, 1),
                      keepdims=True) for u in range(U)]
        r = {}
        for u in range(1, U):
            for v in range(u):
                r[(v, u)] = jnp.sum(jnp.where(masks[u], rows[v], 0.0),
                                    axis=(0, 1), keepdims=True)
        k = [None] * U
        k[0] = ks[0]
        for u in range(1, U):
            p = ks[u]
            for v in range(u):
                p = p * (1.0 - r[(v, u)] * k[v])
            k[u] = p
        upd = 1.0 - rows[0] * k[0]
        for u in range(1, U):
            upd = upd * (1.0 - rows[u] * k[u])
        return keep * upd

    keep88 = lax.fori_loop(0, K // U, nms_group,
                           jnp.ones((8, 128), jnp.float32))
    keep = jnp.concatenate([keep88[s:s + 1, :] for s in range(8)], axis=1)
    out_ref[...] = sel_r * keep


@jax.jit
def kernel(boxes, deltas, scores):
    sp = jnp.pad(scores, (0, NP - N), constant_values=-1.0)
    sr = sp.reshape(1, NP)
    sc = sp.reshape(NP, 1)
    bd = jnp.concatenate(
        [jnp.pad(boxes, ((0, NP - N), (0, 0))).T,
         jnp.pad(deltas, ((0, NP - N), (0, 0))).T], axis=0)

    out = pl.pallas_call(
        _nms_body,
        out_shape=jax.ShapeDtypeStruct((8, KP), jnp.float32),
        scratch_shapes=[pltpu.VMEM((KP, 8, 128), jnp.float32)],
    )(sr, sc, bd)
    return out.T[:K, :5]


# speedup vs baseline: 2.3278x; 1.2895x over previous
import jax
import jax.numpy as jnp
from jax import lax
from jax.experimental import pallas as pl
from jax.experimental.pallas import tpu as pltpu

N = 5000
NP = 5120
K = 1000
KP = 1024
SJ = 32
TN = 512
RB = 128
IOU_THRESHOLD = 0.5
WIN_Y = 512.0
WIN_X = 512.0


def _nms_body(sr_ref, s2_ref, bd_ref, out_ref, s_ref):
    sr = sr_ref[...]
    b = bd_ref[...]

    y1, x1, y2, x2 = b[0:1], b[1:2], b[2:3], b[3:4]
    dy, dx, dh, dw = b[4:5], b[5:6], b[6:7], b[7:8]
    h = y2 - y1
    w = x2 - x1
    cy = y1 + 0.5 * h
    cx = x1 + 0.5 * w
    pcy = dy * h + cy
    pcx = dx * w + cx
    ph = jnp.exp(dh) * h
    pw = jnp.exp(dw) * w
    py1 = jnp.clip(pcy - 0.5 * ph, 0.0, WIN_Y)
    px1 = jnp.clip(pcx - 0.5 * pw, 0.0, WIN_X)
    py2 = jnp.clip(pcy + 0.5 * ph, 0.0, WIN_Y)
    px2 = jnp.clip(pcx + 0.5 * pw, 0.0, WIN_X)
    pred5 = jnp.concatenate(
        [py1, px1, py2, px2, sr, jnp.zeros((3, NP), jnp.float32)], axis=0
    )

    bits2 = lax.bitcast_convert_type(s2_ref[...], jnp.int32)
    fio2 = (lax.broadcasted_iota(jnp.int32, (40, 128), 0) * 128
            + lax.broadcasted_iota(jnp.int32, (40, 128), 1))

    def bs1(_, lohi):
        lo, hi = lohi
        mid = (lo + hi) // 2
        c = jnp.sum((bits2 > mid).astype(jnp.int32))
        big = c >= KP
        return jnp.where(big, mid, lo), jnp.where(big, hi, mid)

    _, tb = lax.fori_loop(0, 31, bs1,
                          (jnp.int32(-1), jnp.int32(0x40000000)))
    m = KP - jnp.sum((bits2 > tb).astype(jnp.int32))
    eq2 = bits2 == tb

    def bs2(_, lohi):
        lo, hi = lohi
        mid = (lo + hi) // 2
        c = jnp.sum((eq2 & (fio2 < mid)).astype(jnp.int32))
        ge = c >= m
        return jnp.where(ge, lo, mid), jnp.where(ge, mid, hi)

    _, it = lax.fori_loop(0, 13, bs2, (jnp.int32(0), jnp.int32(8192)))

    cand2 = (bits2 > tb) | (eq2 & (fio2 < it))
    c2 = cand2.astype(jnp.int32)
    p = c2
    for s in (1, 2, 4, 8, 16, 32, 64):
        p = p + jnp.concatenate(
            [jnp.zeros((40, s), jnp.int32), p[:, :128 - s]], axis=1)
    row_incl = p[:, 127:128]
    q = row_incl
    for s in (1, 2, 4, 8, 16, 32):
        q = q + jnp.concatenate(
            [jnp.zeros((s, 1), jnp.int32), q[:40 - s, :]], axis=0)
    pos2 = p - c2 + (q - row_incl)

    posr = jnp.concatenate([pos2[r:r + 1, :] for r in range(40)], axis=1)
    candr = jnp.concatenate([c2[r:r + 1, :] for r in range(40)], axis=1)

    kio = lax.broadcasted_iota(jnp.int32, (KP, 1), 0)
    dn = (((1,), (1,)), ((), ()))
    csel_r = jnp.zeros((8, KP), jnp.float32)
    csel_c = jnp.zeros((KP, 8), jnp.float32)
    for t in range(NP // TN):
        sl = slice(t * TN, (t + 1) * TN)
        oh = jnp.where((candr[:, sl] > 0) & (posr[:, sl] == kio), 1.0, 0.0)
        pr = pred5[:, sl]
        csel_r = csel_r + lax.dot_general(
            pr, oh, dn, precision=lax.Precision.HIGHEST,
            preferred_element_type=jnp.float32)
        csel_c = csel_c + lax.dot_general(
            oh, pr, dn, precision=lax.Precision.HIGHEST,
            preferred_element_type=jnp.float32)

    csr = csel_r[4:5, :]
    lioK = lax.broadcasted_iota(jnp.int32, (1, KP), 1)

    rank = jnp.zeros((1, KP), jnp.float32)
    for jt in range(KP // SJ):
        sct = csel_c[jt * SJ:(jt + 1) * SJ, 4:5]
        jidx = jt * SJ + lax.broadcasted_iota(jnp.int32, (SJ, 1), 0)
        beats = (sct > csr) | ((sct == csr) & (jidx < lioK))
        rank = rank + jnp.sum(
            jnp.where(beats, 1.0, 0.0), axis=0, keepdims=True)

    kiof = kio.astype(jnp.float32)
    Q = jnp.where(rank == kiof, 1.0, 0.0)
    sel_r = lax.dot_general(csel_r, Q, dn,
                            precision=lax.Precision.HIGHEST,
                            preferred_element_type=jnp.float32)
    sel_c = lax.dot_general(Q, csel_r, dn,
                            precision=lax.Precision.HIGHEST,
                            preferred_element_type=jnp.float32)

    def to88(v):
        return jnp.concatenate(
            [v[:, s * 128:(s + 1) * 128] for s in range(8)],
            axis=0).reshape(1, 8, 128)

    y1r, x1r, y2r, x2r = (to88(sel_r[c:c + 1]) for c in range(4))
    area_r = jnp.maximum(y2r - y1r, 0.0) * jnp.maximum(x2r - x1r, 0.0)
    lio3 = (lax.broadcasted_iota(jnp.int32, (1, 8, 128), 1) * 128
            + lax.broadcasted_iota(jnp.int32, (1, 8, 128), 2))
    for rb in range(KP // RB):
        sl = slice(rb * RB, (rb + 1) * RB)
        y1c = sel_c[sl, 0:1].reshape(RB, 1, 1)
        x1c = sel_c[sl, 1:2].reshape(RB, 1, 1)
        y2c = sel_c[sl, 2:3].reshape(RB, 1, 1)
        x2c = sel_c[sl, 3:4].reshape(RB, 1, 1)
        area_c = jnp.maximum(y2c - y1c, 0.0) * jnp.maximum(x2c - x1c, 0.0)
        yy1 = jnp.maximum(y1c, y1r)
        xx1 = jnp.maximum(x1c, x1r)
        yy2 = jnp.minimum(y2c, y2r)
        xx2 = jnp.minimum(x2c, x2r)
        inter = jnp.maximum(yy2 - yy1, 0.0) * jnp.maximum(xx2 - xx1, 0.0)
        union = area_c + area_r - inter
        iou = inter / (union + 1e-6)
        sio = lax.broadcasted_iota(jnp.int32, (RB, 1, 1), 0) + rb * RB
        s_ref[sl] = jnp.where((iou > IOU_THRESHOLD) & (lio3 > sio), 1.0, 0.0)

    iota88 = lio3.reshape(8, 128)
    U = 8

    def nms_group(g, keep):
        base = g * U
        rows3 = s_ref[pl.ds(base, U)]
        rows = [rows3[u].reshape(8, 128) for u in range(U)]
        masks = [iota88 == (base + u) for u in range(U)]
        ks = [jnp.sum(jnp.where(masks[u], keep, 0.0), axis=(0, 1),
                      keepdims=True) for u in range(U)]
        r = {}
        for u in range(1, U):
            for v in range(u):
                r[(v, u)] = jnp.sum(jnp.where(masks[u], rows[v], 0.0),
                                    axis=(0, 1), keepdims=True)
        k = [None] * U
        k[0] = ks[0]
        for u in range(1, U):
            pp = ks[u]
            for v in range(u):
                pp = pp * (1.0 - r[(v, u)] * k[v])
            k[u] = pp
        upd = 1.0 - rows[0] * k[0]
        for u in range(1, U):
            upd = upd * (1.0 - rows[u] * k[u])
        return keep * upd

    keep88 = lax.fori_loop(0, K // U, nms_group,
                           jnp.ones((8, 128), jnp.float32))
    keep = jnp.concatenate([keep88[s:s + 1, :] for s in range(8)], axis=1)
    out_ref[...] = sel_r * keep


@jax.jit
def kernel(boxes, deltas, scores):
    sp = jnp.pad(scores, (0, NP - N), constant_values=0.0)
    sr = sp.reshape(1, NP)
    s2 = sp.reshape(40, 128)
    bd = jnp.concatenate(
        [jnp.pad(boxes, ((0, NP - N), (0, 0))).T,
         jnp.pad(deltas, ((0, NP - N), (0, 0))).T], axis=0)

    out = pl.pallas_call(
        _nms_body,
        out_shape=jax.ShapeDtypeStruct((8, KP), jnp.float32),
        scratch_shapes=[pltpu.VMEM((KP, 8, 128), jnp.float32)],
    )(sr, s2, bd)
    return out.T[:K, :5]


# speedup vs baseline: 2.3692x; 1.0178x over previous
import jax
import jax.numpy as jnp
from jax import lax
from jax.experimental import pallas as pl
from jax.experimental.pallas import tpu as pltpu

N = 5000
NP = 5120
K = 1000
KP = 1024
SJ = 32
TN = 512
RB = 128
IOU_THRESHOLD = 0.5
WIN_Y = 512.0
WIN_X = 512.0


def _nms_body(sr_ref, s2_ref, bd_ref, out_ref, s_ref):
    sr = sr_ref[...]
    b = bd_ref[...]

    y1, x1, y2, x2 = b[0:1], b[1:2], b[2:3], b[3:4]
    dy, dx, dh, dw = b[4:5], b[5:6], b[6:7], b[7:8]
    h = y2 - y1
    w = x2 - x1
    cy = y1 + 0.5 * h
    cx = x1 + 0.5 * w
    pcy = dy * h + cy
    pcx = dx * w + cx
    ph = jnp.exp(dh) * h
    pw = jnp.exp(dw) * w
    py1 = jnp.clip(pcy - 0.5 * ph, 0.0, WIN_Y)
    px1 = jnp.clip(pcx - 0.5 * pw, 0.0, WIN_X)
    py2 = jnp.clip(pcy + 0.5 * ph, 0.0, WIN_Y)
    px2 = jnp.clip(pcx + 0.5 * pw, 0.0, WIN_X)
    pred5 = jnp.concatenate(
        [py1, px1, py2, px2, sr, jnp.zeros((3, NP), jnp.float32)], axis=0
    )

    bits2 = lax.bitcast_convert_type(s2_ref[...], jnp.int32)
    fio2 = (lax.broadcasted_iota(jnp.int32, (40, 128), 0) * 128
            + lax.broadcasted_iota(jnp.int32, (40, 128), 1))

    def bs1(_, lohi):
        lo, hi = lohi
        mid = (lo + hi) // 2
        c = jnp.sum((bits2 > mid).astype(jnp.int32))
        big = c >= KP
        return jnp.where(big, mid, lo), jnp.where(big, hi, mid)

    _, tb = lax.fori_loop(0, 31, bs1,
                          (jnp.int32(-1), jnp.int32(0x40000000)))
    m = KP - jnp.sum((bits2 > tb).astype(jnp.int32))
    eq2 = bits2 == tb

    def bs2(_, lohi):
        lo, hi = lohi
        mid = (lo + hi) // 2
        c = jnp.sum((eq2 & (fio2 < mid)).astype(jnp.int32))
        ge = c >= m
        return jnp.where(ge, lo, mid), jnp.where(ge, mid, hi)

    _, it = lax.fori_loop(0, 13, bs2, (jnp.int32(0), jnp.int32(8192)))

    cand2 = (bits2 > tb) | (eq2 & (fio2 < it))
    c2 = cand2.astype(jnp.int32)
    p = c2
    for s in (1, 2, 4, 8, 16, 32, 64):
        p = p + jnp.concatenate(
            [jnp.zeros((40, s), jnp.int32), p[:, :128 - s]], axis=1)
    row_incl = p[:, 127:128]
    q = row_incl
    for s in (1, 2, 4, 8, 16, 32):
        q = q + jnp.concatenate(
            [jnp.zeros((s, 1), jnp.int32), q[:40 - s, :]], axis=0)
    pos2 = p - c2 + (q - row_incl)

    posr = jnp.concatenate([pos2[r:r + 1, :] for r in range(40)], axis=1)
    candr = jnp.concatenate([c2[r:r + 1, :] for r in range(40)], axis=1)

    kio = lax.broadcasted_iota(jnp.int32, (KP, 1), 0)
    dn = (((1,), (1,)), ((), ()))
    csel_r = jnp.zeros((8, KP), jnp.float32)
    csel_c = jnp.zeros((KP, 8), jnp.float32)
    for t in range(NP // TN):
        sl = slice(t * TN, (t + 1) * TN)
        oh = jnp.where((candr[:, sl] > 0) & (posr[:, sl] == kio), 1.0, 0.0)
        pr = pred5[:, sl]
        csel_r = csel_r + lax.dot_general(
            pr, oh, dn, precision=lax.Precision.HIGHEST,
            preferred_element_type=jnp.float32)
        csel_c = csel_c + lax.dot_general(
            oh, pr, dn, precision=lax.Precision.HIGHEST,
            preferred_element_type=jnp.float32)

    csr = csel_r[4:5, :]
    lioK = lax.broadcasted_iota(jnp.int32, (1, KP), 1)

    rank = jnp.zeros((1, KP), jnp.float32)
    for jt in range(KP // SJ):
        sct = csel_c[jt * SJ:(jt + 1) * SJ, 4:5]
        jidx = jt * SJ + lax.broadcasted_iota(jnp.int32, (SJ, 1), 0)
        beats = (sct > csr) | ((sct == csr) & (jidx < lioK))
        rank = rank + jnp.sum(
            jnp.where(beats, 1.0, 0.0), axis=0, keepdims=True)

    kiof = kio.astype(jnp.float32)
    Q = jnp.where(rank == kiof, 1.0, 0.0)
    sel_r = lax.dot_general(csel_r, Q, dn,
                            precision=lax.Precision.HIGHEST,
                            preferred_element_type=jnp.float32)
    sel_c = lax.dot_general(Q, csel_r, dn,
                            precision=lax.Precision.HIGHEST,
                            preferred_element_type=jnp.float32)

    def to88(v):
        return jnp.concatenate(
            [v[:, s * 128:(s + 1) * 128] for s in range(8)],
            axis=0).reshape(1, 8, 128)

    y1r, x1r, y2r, x2r = (to88(sel_r[c:c + 1]) for c in range(4))
    area_r = jnp.maximum(y2r - y1r, 0.0) * jnp.maximum(x2r - x1r, 0.0)
    lio3 = (lax.broadcasted_iota(jnp.int32, (1, 8, 128), 1) * 128
            + lax.broadcasted_iota(jnp.int32, (1, 8, 128), 2))
    for rb in range(KP // RB):
        sl = slice(rb * RB, (rb + 1) * RB)
        y1c = sel_c[sl, 0:1].reshape(RB, 1, 1)
        x1c = sel_c[sl, 1:2].reshape(RB, 1, 1)
        y2c = sel_c[sl, 2:3].reshape(RB, 1, 1)
        x2c = sel_c[sl, 3:4].reshape(RB, 1, 1)
        area_c = jnp.maximum(y2c - y1c, 0.0) * jnp.maximum(x2c - x1c, 0.0)
        yy1 = jnp.maximum(y1c, y1r)
        xx1 = jnp.maximum(x1c, x1r)
        yy2 = jnp.minimum(y2c, y2r)
        xx2 = jnp.minimum(x2c, x2r)
        inter = jnp.maximum(yy2 - yy1, 0.0) * jnp.maximum(xx2 - xx1, 0.0)
        union = area_c + area_r - inter
        iou = inter / (union + 1e-6)
        sio = lax.broadcasted_iota(jnp.int32, (RB, 1, 1), 0) + rb * RB
        s_ref[sl] = jnp.where((iou > IOU_THRESHOLD) & (lio3 > sio), 1.0, 0.0)

    iota88 = lio3.reshape(8, 128)
    U = 8
    wt_all = (1 << (iota88 & 7)).astype(jnp.float32)

    def nms_group(g, keep):
        base = g * U
        rows3 = s_ref[pl.ds(base, U)]
        rows = [rows3[u].reshape(8, 128) for u in range(U)]
        gm = (iota88 >= base) & (iota88 < base + U)
        wt = jnp.where(gm, wt_all, 0.0)
        kbits = jnp.sum(keep * wt, axis=(0, 1),
                        keepdims=True).astype(jnp.int32)
        rbits = [jnp.sum(rows[v] * wt, axis=(0, 1),
                         keepdims=True).astype(jnp.int32)
                 for v in range(U - 1)]
        k = [None] * U
        k[0] = kbits & 1
        for u in range(1, U):
            ku = (kbits >> u) & 1
            for v in range(u):
                ku = ku * (1 - ((rbits[v] >> u) & 1) * k[v])
            k[u] = ku
        upd = 1.0 - rows[0] * k[0].astype(jnp.float32)
        for u in range(1, U):
            upd = upd * (1.0 - rows[u] * k[u].astype(jnp.float32))
        return keep * upd

    keep88 = lax.fori_loop(0, K // U, nms_group,
                           jnp.ones((8, 128), jnp.float32))
    keep = jnp.concatenate([keep88[s:s + 1, :] for s in range(8)], axis=1)
    out_ref[...] = sel_r * keep


@jax.jit
def kernel(boxes, deltas, scores):
    sp = jnp.pad(scores, (0, NP - N), constant_values=0.0)
    sr = sp.reshape(1, NP)
    s2 = sp.reshape(40, 128)
    bd = jnp.concatenate(
        [jnp.pad(boxes, ((0, NP - N), (0, 0))).T,
         jnp.pad(deltas, ((0, NP - N), (0, 0))).T], axis=0)

    out = pl.pallas_call(
        _nms_body,
        out_shape=jax.ShapeDtypeStruct((8, KP), jnp.float32),
        scratch_shapes=[pltpu.VMEM((KP, 8, 128), jnp.float32)],
    )(sr, s2, bd)
    return out.T[:K, :5]


# speedup vs baseline: 3.3374x; 1.4086x over previous
import jax
import jax.numpy as jnp
from jax import lax
from jax.experimental import pallas as pl
from jax.experimental.pallas import tpu as pltpu

N = 5000
NP = 5120
K = 1000
KP = 1024
SJ = 32
TN = 512
RB = 128
IOU_THRESHOLD = 0.5
WIN_Y = 512.0
WIN_X = 512.0


def _dot3(a_f32, b01, dn):
    hi = a_f32.astype(jnp.bfloat16)
    r1 = a_f32 - hi.astype(jnp.float32)
    mid = r1.astype(jnp.bfloat16)
    lo = (r1 - mid.astype(jnp.float32)).astype(jnp.bfloat16)
    bb = b01.astype(jnp.bfloat16)
    out = lax.dot_general(hi, bb, dn, preferred_element_type=jnp.float32)
    out = out + lax.dot_general(mid, bb, dn,
                                preferred_element_type=jnp.float32)
    out = out + lax.dot_general(lo, bb, dn,
                                preferred_element_type=jnp.float32)
    return out


def _dot3r(b01, a_f32, dn):
    hi = a_f32.astype(jnp.bfloat16)
    r1 = a_f32 - hi.astype(jnp.float32)
    mid = r1.astype(jnp.bfloat16)
    lo = (r1 - mid.astype(jnp.float32)).astype(jnp.bfloat16)
    bb = b01.astype(jnp.bfloat16)
    out = lax.dot_general(bb, hi, dn, preferred_element_type=jnp.float32)
    out = out + lax.dot_general(bb, mid, dn,
                                preferred_element_type=jnp.float32)
    out = out + lax.dot_general(bb, lo, dn,
                                preferred_element_type=jnp.float32)
    return out


def _nms_body(sr_ref, s2_ref, bd_ref, out_ref, s_ref):
    sr = sr_ref[...]
    b = bd_ref[...]

    y1, x1, y2, x2 = b[0:1], b[1:2], b[2:3], b[3:4]
    dy, dx, dh, dw = b[4:5], b[5:6], b[6:7], b[7:8]
    h = y2 - y1
    w = x2 - x1
    cy = y1 + 0.5 * h
    cx = x1 + 0.5 * w
    pcy = dy * h + cy
    pcx = dx * w + cx
    ph = jnp.exp(dh) * h
    pw = jnp.exp(dw) * w
    py1 = jnp.clip(pcy - 0.5 * ph, 0.0, WIN_Y)
    px1 = jnp.clip(pcx - 0.5 * pw, 0.0, WIN_X)
    py2 = jnp.clip(pcy + 0.5 * ph, 0.0, WIN_Y)
    px2 = jnp.clip(pcx + 0.5 * pw, 0.0, WIN_X)
    pred5 = jnp.concatenate(
        [py1, px1, py2, px2, sr, jnp.zeros((3, NP), jnp.float32)], axis=0
    )

    bits2 = lax.bitcast_convert_type(s2_ref[...], jnp.int32)
    fio2 = (lax.broadcasted_iota(jnp.int32, (40, 128), 0) * 128
            + lax.broadcasted_iota(jnp.int32, (40, 128), 1))

    def bs1(_, lohi):
        lo, hi = lohi
        mid = (lo + hi) // 2
        c = jnp.sum((bits2 > mid).astype(jnp.int32))
        big = c >= KP
        return jnp.where(big, mid, lo), jnp.where(big, hi, mid)

    _, tb = lax.fori_loop(0, 31, bs1,
                          (jnp.int32(-1), jnp.int32(0x40000000)))
    m = KP - jnp.sum((bits2 > tb).astype(jnp.int32))
    eq2 = bits2 == tb

    def bs2(_, lohi):
        lo, hi = lohi
        mid = (lo + hi) // 2
        c = jnp.sum((eq2 & (fio2 < mid)).astype(jnp.int32))
        ge = c >= m
        return jnp.where(ge, lo, mid), jnp.where(ge, mid, hi)

    _, it = lax.fori_loop(0, 13, bs2, (jnp.int32(0), jnp.int32(8192)))

    cand2 = (bits2 > tb) | (eq2 & (fio2 < it))
    c2 = cand2.astype(jnp.int32)
    p = c2
    for s in (1, 2, 4, 8, 16, 32, 64):
        p = p + jnp.concatenate(
            [jnp.zeros((40, s), jnp.int32), p[:, :128 - s]], axis=1)
    row_incl = p[:, 127:128]
    q = row_incl
    for s in (1, 2, 4, 8, 16, 32):
        q = q + jnp.concatenate(
            [jnp.zeros((s, 1), jnp.int32), q[:40 - s, :]], axis=0)
    pos2 = p - c2 + (q - row_incl)

    posr = jnp.concatenate([pos2[r:r + 1, :] for r in range(40)], axis=1)
    candr = jnp.concatenate([c2[r:r + 1, :] for r in range(40)], axis=1)

    kio = lax.broadcasted_iota(jnp.int32, (KP, 1), 0)
    dn = (((1,), (1,)), ((), ()))
    csel_r = jnp.zeros((8, KP), jnp.float32)
    csel_c = jnp.zeros((KP, 8), jnp.float32)
    for t in range(NP // TN):
        sl = slice(t * TN, (t + 1) * TN)
        oh = jnp.where((candr[:, sl] > 0) & (posr[:, sl] == kio), 1.0, 0.0)
        pr = pred5[:, sl]
        csel_r = csel_r + _dot3(pr, oh, dn)
        csel_c = csel_c + _dot3r(oh, pr, dn)

    csr = csel_r[4:5, :]
    lioK = lax.broadcasted_iota(jnp.int32, (1, KP), 1)

    rank = jnp.zeros((1, KP), jnp.float32)
    for jt in range(KP // SJ):
        sct = csel_c[jt * SJ:(jt + 1) * SJ, 4:5]
        jidx = jt * SJ + lax.broadcasted_iota(jnp.int32, (SJ, 1), 0)
        beats = (sct > csr) | ((sct == csr) & (jidx < lioK))
        rank = rank + jnp.sum(
            jnp.where(beats, 1.0, 0.0), axis=0, keepdims=True)

    kiof = kio.astype(jnp.float32)
    Q = jnp.where(rank == kiof, 1.0, 0.0)
    sel_r = _dot3(csel_r, Q, dn)
    sel_c = _dot3r(Q, csel_r, dn)

    def to88(v):
        return jnp.concatenate(
            [v[:, s * 128:(s + 1) * 128] for s in range(8)],
            axis=0).reshape(1, 8, 128)

    y1r, x1r, y2r, x2r = (to88(sel_r[c:c + 1]) for c in range(4))
    area_r = jnp.maximum(y2r - y1r, 0.0) * jnp.maximum(x2r - x1r, 0.0)
    lio3 = (lax.broadcasted_iota(jnp.int32, (1, 8, 128), 1) * 128
            + lax.broadcasted_iota(jnp.int32, (1, 8, 128), 2))
    for rb in range(KP // RB):
        sl = slice(rb * RB, (rb + 1) * RB)
        y1c = sel_c[sl, 0:1].reshape(RB, 1, 1)
        x1c = sel_c[sl, 1:2].reshape(RB, 1, 1)
        y2c = sel_c[sl, 2:3].reshape(RB, 1, 1)
        x2c = sel_c[sl, 3:4].reshape(RB, 1, 1)
        area_c = jnp.maximum(y2c - y1c, 0.0) * jnp.maximum(x2c - x1c, 0.0)
        yy1 = jnp.maximum(y1c, y1r)
        xx1 = jnp.maximum(x1c, x1r)
        yy2 = jnp.minimum(y2c, y2r)
        xx2 = jnp.minimum(x2c, x2r)
        inter = jnp.maximum(yy2 - yy1, 0.0) * jnp.maximum(xx2 - xx1, 0.0)
        union = area_c + area_r - inter
        iou = inter / (union + 1e-6)
        sio = lax.broadcasted_iota(jnp.int32, (RB, 1, 1), 0) + rb * RB
        s_ref[sl] = jnp.where((iou > IOU_THRESHOLD) & (lio3 > sio), 1.0, 0.0)

    iota88 = lio3.reshape(8, 128)
    U = 8
    wt_all = (1 << (iota88 & 7)).astype(jnp.float32)

    def nms_group(g, keep):
        base = g * U
        rows3 = s_ref[pl.ds(base, U)]
        rows = [rows3[u].reshape(8, 128) for u in range(U)]
        gm = (iota88 >= base) & (iota88 < base + U)
        wt = jnp.where(gm, wt_all, 0.0)
        kbits = jnp.sum(keep * wt, axis=(0, 1),
                        keepdims=True).astype(jnp.int32)
        rbits = [jnp.sum(rows[v] * wt, axis=(0, 1),
                         keepdims=True).astype(jnp.int32)
                 for v in range(U - 1)]
        k = [None] * U
        k[0] = kbits & 1
        for u in range(1, U):
            ku = (kbits >> u) & 1
            for v in range(u):
                ku = ku * (1 - ((rbits[v] >> u) & 1) * k[v])
            k[u] = ku
        upd = 1.0 - rows[0] * k[0].astype(jnp.float32)
        for u in range(1, U):
            upd = upd * (1.0 - rows[u] * k[u].astype(jnp.float32))
        return keep * upd

    keep88 = lax.fori_loop(0, K // U, nms_group,
                           jnp.ones((8, 128), jnp.float32))
    keep = jnp.concatenate([keep88[s:s + 1, :] for s in range(8)], axis=1)
    out_ref[...] = sel_r * keep


@jax.jit
def kernel(boxes, deltas, scores):
    sp = jnp.pad(scores, (0, NP - N), constant_values=0.0)
    sr = sp.reshape(1, NP)
    s2 = sp.reshape(40, 128)
    bd = jnp.concatenate(
        [jnp.pad(boxes, ((0, NP - N), (0, 0))).T,
         jnp.pad(deltas, ((0, NP - N), (0, 0))).T], axis=0)

    out = pl.pallas_call(
        _nms_body,
        out_shape=jax.ShapeDtypeStruct((8, KP), jnp.float32),
        scratch_shapes=[pltpu.VMEM((KP, 8, 128), jnp.float32)],
    )(sr, s2, bd)
    return out.T[:K, :5]


# speedup vs baseline: 3.6167x; 1.0837x over previous
import jax
import jax.numpy as jnp
from jax import lax
from jax.experimental import pallas as pl
from jax.experimental.pallas import tpu as pltpu

N = 5000
NP = 5120
K = 1000
KP = 1024
SJ = 32
TN = 512
RB = 128
IOU_THRESHOLD = 0.5
WIN_Y = 512.0
WIN_X = 512.0


def _dot3(a_f32, b01, dn):
    hi = a_f32.astype(jnp.bfloat16)
    r1 = a_f32 - hi.astype(jnp.float32)
    mid = r1.astype(jnp.bfloat16)
    lo = (r1 - mid.astype(jnp.float32)).astype(jnp.bfloat16)
    bb = b01.astype(jnp.bfloat16)
    out = lax.dot_general(hi, bb, dn, preferred_element_type=jnp.float32)
    out = out + lax.dot_general(mid, bb, dn,
                                preferred_element_type=jnp.float32)
    out = out + lax.dot_general(lo, bb, dn,
                                preferred_element_type=jnp.float32)
    return out


def _dot3r(b01, a_f32, dn):
    hi = a_f32.astype(jnp.bfloat16)
    r1 = a_f32 - hi.astype(jnp.float32)
    mid = r1.astype(jnp.bfloat16)
    lo = (r1 - mid.astype(jnp.float32)).astype(jnp.bfloat16)
    bb = b01.astype(jnp.bfloat16)
    out = lax.dot_general(bb, hi, dn, preferred_element_type=jnp.float32)
    out = out + lax.dot_general(bb, mid, dn,
                                preferred_element_type=jnp.float32)
    out = out + lax.dot_general(bb, lo, dn,
                                preferred_element_type=jnp.float32)
    return out


def _nms_body(sr_ref, s2_ref, bd_ref, out_ref, s_ref):
    sr = sr_ref[...]
    b = bd_ref[...]

    y1, x1, y2, x2 = b[0:1], b[1:2], b[2:3], b[3:4]
    dy, dx, dh, dw = b[4:5], b[5:6], b[6:7], b[7:8]
    h = y2 - y1
    w = x2 - x1
    cy = y1 + 0.5 * h
    cx = x1 + 0.5 * w
    pcy = dy * h + cy
    pcx = dx * w + cx
    ph = jnp.exp(dh) * h
    pw = jnp.exp(dw) * w
    py1 = jnp.clip(pcy - 0.5 * ph, 0.0, WIN_Y)
    px1 = jnp.clip(pcx - 0.5 * pw, 0.0, WIN_X)
    py2 = jnp.clip(pcy + 0.5 * ph, 0.0, WIN_Y)
    px2 = jnp.clip(pcx + 0.5 * pw, 0.0, WIN_X)
    pred5 = jnp.concatenate(
        [py1, px1, py2, px2, sr, jnp.zeros((3, NP), jnp.float32)], axis=0
    )

    bits2 = lax.bitcast_convert_type(s2_ref[...], jnp.int32)
    fio2 = (lax.broadcasted_iota(jnp.int32, (40, 128), 0) * 128
            + lax.broadcasted_iota(jnp.int32, (40, 128), 1))

    def bs1(_, lohi):
        lo, hi = lohi
        mid = (lo + hi) // 2
        c = jnp.sum((bits2 > mid).astype(jnp.int32))
        big = c >= KP
        return jnp.where(big, mid, lo), jnp.where(big, hi, mid)

    _, tb = lax.fori_loop(0, 31, bs1,
                          (jnp.int32(-1), jnp.int32(0x40000000)))
    m = KP - jnp.sum((bits2 > tb).astype(jnp.int32))
    eq2 = bits2 == tb

    def bs2(_, lohi):
        lo, hi = lohi
        mid = (lo + hi) // 2
        c = jnp.sum((eq2 & (fio2 < mid)).astype(jnp.int32))
        ge = c >= m
        return jnp.where(ge, lo, mid), jnp.where(ge, mid, hi)

    _, it = lax.fori_loop(0, 13, bs2, (jnp.int32(0), jnp.int32(8192)))

    cand2 = (bits2 > tb) | (eq2 & (fio2 < it))
    c2 = cand2.astype(jnp.int32)
    p = c2
    for s in (1, 2, 4, 8, 16, 32, 64):
        p = p + jnp.concatenate(
            [jnp.zeros((40, s), jnp.int32), p[:, :128 - s]], axis=1)
    row_incl = p[:, 127:128]
    q = row_incl
    for s in (1, 2, 4, 8, 16, 32):
        q = q + jnp.concatenate(
            [jnp.zeros((s, 1), jnp.int32), q[:40 - s, :]], axis=0)
    pos2 = p - c2 + (q - row_incl)

    posr = jnp.concatenate([pos2[r:r + 1, :] for r in range(40)], axis=1)
    candr = jnp.concatenate([c2[r:r + 1, :] for r in range(40)], axis=1)

    kio = lax.broadcasted_iota(jnp.int32, (KP, 1), 0)
    dn = (((1,), (1,)), ((), ()))
    csel_r = jnp.zeros((8, KP), jnp.float32)
    csel_c = jnp.zeros((KP, 8), jnp.float32)
    for t in range(NP // TN):
        sl = slice(t * TN, (t + 1) * TN)
        oh = jnp.where((candr[:, sl] > 0) & (posr[:, sl] == kio), 1.0, 0.0)
        pr = pred5[:, sl]
        csel_r = csel_r + _dot3(pr, oh, dn)
        csel_c = csel_c + _dot3r(oh, pr, dn)

    csr = csel_r[4:5, :]
    lioK = lax.broadcasted_iota(jnp.int32, (1, KP), 1)

    rank = jnp.zeros((1, KP), jnp.float32)
    for jt in range(KP // SJ):
        sct = csel_c[jt * SJ:(jt + 1) * SJ, 4:5]
        jidx = jt * SJ + lax.broadcasted_iota(jnp.int32, (SJ, 1), 0)
        beats = (sct > csr) | ((sct == csr) & (jidx < lioK))
        rank = rank + jnp.sum(
            jnp.where(beats, 1.0, 0.0), axis=0, keepdims=True)

    kiof = kio.astype(jnp.float32)
    Q = jnp.where(rank == kiof, 1.0, 0.0)
    sel_r = _dot3(csel_r, Q, dn)
    sel_c = _dot3r(Q, csel_r, dn)

    def to88(v):
        return jnp.concatenate(
            [v[:, s * 128:(s + 1) * 128] for s in range(8)],
            axis=0).reshape(1, 8, 128)

    y1r, x1r, y2r, x2r = (to88(sel_r[c:c + 1]) for c in range(4))
    area_r = jnp.maximum(y2r - y1r, 0.0) * jnp.maximum(x2r - x1r, 0.0)
    lio3 = (lax.broadcasted_iota(jnp.int32, (1, 8, 128), 1) * 128
            + lax.broadcasted_iota(jnp.int32, (1, 8, 128), 2))
    for rb in range(KP // RB):
        sl = slice(rb * RB, (rb + 1) * RB)
        y1c = sel_c[sl, 0:1].reshape(RB, 1, 1)
        x1c = sel_c[sl, 1:2].reshape(RB, 1, 1)
        y2c = sel_c[sl, 2:3].reshape(RB, 1, 1)
        x2c = sel_c[sl, 3:4].reshape(RB, 1, 1)
        area_c = jnp.maximum(y2c - y1c, 0.0) * jnp.maximum(x2c - x1c, 0.0)
        yy1 = jnp.maximum(y1c, y1r)
        xx1 = jnp.maximum(x1c, x1r)
        yy2 = jnp.minimum(y2c, y2r)
        xx2 = jnp.minimum(x2c, x2r)
        inter = jnp.maximum(yy2 - yy1, 0.0) * jnp.maximum(xx2 - xx1, 0.0)
        union = area_c + area_r - inter
        iou = inter / (union + 1e-6)
        sio = lax.broadcasted_iota(jnp.int32, (RB, 1, 1), 0) + rb * RB
        s_ref[sl] = jnp.where((iou > IOU_THRESHOLD) & (lio3 > sio), 1.0, 0.0)

    iota88 = lio3.reshape(8, 128)
    U = 16
    wt_all = (1 << (iota88 & 15)).astype(jnp.float32)

    def nms_group(g, keep):
        base = g * U
        rows3 = s_ref[pl.ds(base, U)]
        rows = [rows3[u].reshape(8, 128) for u in range(U)]
        gm = (iota88 >= base) & (iota88 < base + U)
        wt = jnp.where(gm, wt_all, 0.0)
        kbits = jnp.sum(keep * wt, axis=(0, 1),
                        keepdims=True).astype(jnp.int32)
        rbits = [jnp.sum(rows[v] * wt, axis=(0, 1),
                         keepdims=True).astype(jnp.int32)
                 for v in range(U - 1)]
        k = [None] * U
        k[0] = kbits & 1
        for u in range(1, U):
            ku = (kbits >> u) & 1
            for v in range(u):
                ku = ku * (1 - ((rbits[v] >> u) & 1) * k[v])
            k[u] = ku
        upd = 1.0 - rows[0] * k[0].astype(jnp.float32)
        for u in range(1, U):
            upd = upd * (1.0 - rows[u] * k[u].astype(jnp.float32))
        return keep * upd

    keep88 = lax.fori_loop(0, (K + U - 1) // U, nms_group,
                           jnp.ones((8, 128), jnp.float32))
    keep = jnp.concatenate([keep88[s:s + 1, :] for s in range(8)], axis=1)
    out_ref[...] = sel_r * keep


@jax.jit
def kernel(boxes, deltas, scores):
    sp = jnp.pad(scores, (0, NP - N), constant_values=0.0)
    sr = sp.reshape(1, NP)
    s2 = sp.reshape(40, 128)
    bd = jnp.concatenate(
        [jnp.pad(boxes, ((0, NP - N), (0, 0))).T,
         jnp.pad(deltas, ((0, NP - N), (0, 0))).T], axis=0)

    out = pl.pallas_call(
        _nms_body,
        out_shape=jax.ShapeDtypeStruct((8, KP), jnp.float32),
        scratch_shapes=[pltpu.VMEM((KP, 8, 128), jnp.float32)],
    )(sr, s2, bd)
    return out.T[:K, :5]
